# Initial kernel scaffold; baseline (speedup 1.0000x reference)
import jax
import jax.numpy as jnp
from jax.experimental import pallas as pl


def kernel(pred, x, epoch):
    raise NotImplementedError("write your pallas kernel here")


# speedup vs baseline: 26.7130x; 26.7130x over previous
import functools

import jax
import jax.numpy as jnp
from jax.experimental import pallas as pl

_N = 7
_NV = _N * _N
_CTR = _NV // 2
_TOPK = 16
_TH = 8


def _body(h, w, hi_ref, lo_ref, pred_ref, out_ref):
    f32 = jnp.float32
    t = pl.program_id(1)
    first = (pl.program_id(0) == 0) & (t == 0)

    xb = jnp.concatenate([hi_ref[0], lo_ref[0]], axis=1)
    p16 = pred_ref[0, pl.ds(8 * (t + 1), 16), :]
    p9 = p16[3:12, :]
    predt = p9[0:8, 4:4 + w]

    riota = jax.lax.broadcasted_iota(jnp.int32, (_TH, w), 0).astype(f32)
    ciota = jax.lax.broadcasted_iota(jnp.int32, (_TH, w), 1).astype(f32)
    y0f = (8 + 8 * t).astype(f32)
    yy = riota + y0f

    wa = {}
    wb = {}
    for d in range(-3, 4):
        df = float(d)
        sy = jnp.clip(yy + df * predt, 0.0, float(h - 1))
        f0 = jnp.floor(sy)
        wyf = sy - f0
        rel0 = f0 - yy
        rel1 = jnp.minimum(f0 + 1.0, float(h - 1)) - yy
        wa[d] = [
            (rel0 == float(a)).astype(f32) * (1.0 - wyf)
            + (rel1 == float(a)).astype(f32) * wyf
            for a in range(-3, 5)
        ]
        sx = jnp.clip(ciota + df * predt, 0.0, float(w - 1))
        g0 = jnp.floor(sx)
        wxf = sx - g0
        cel0 = g0 - ciota
        cel1 = jnp.minimum(g0 + 1.0, float(w - 1)) - ciota
        wb[d] = [
            (cel0 == float(bb)).astype(f32) * (1.0 - wxf)
            + (cel1 == float(bb)).astype(f32) * wxf
            for bb in range(-3, 5)
        ]

    colmask = ((ciota >= 8.0) & (ciota < float(w - 8))).astype(f32)

    cs = f32(0)
    for ch in range(3):
        ctr = xb[ch * _NV + _CTR, 3:11, 4:4 + w]
        tot = jnp.zeros((_TH, w), f32)
        keys = []
        for vi in range(_NV):
            du = vi // _N - 3
            dv = vi % _N - 3
            acc = jnp.zeros((_TH, w), f32)
            for ai, a in enumerate(range(-3, 5)):
                row = xb[ch * _NV + vi, 3 + a:11 + a, :]
                inner = wb[dv][0] * row[:, 1:1 + w]
                for bi, bb in enumerate(range(-2, 5)):
                    inner = inner + wb[dv][bi + 1] * row[:, 4 + bb:4 + bb + w]
                acc = acc + wa[du][ai] * inner
            diff = jnp.abs(acc - ctr)
            tot = tot + diff
            bits = jax.lax.bitcast_convert_type(diff, jnp.int32)
            keys.append((bits & jnp.int32(-64)) | jnp.int32(vi))
        kk = jnp.stack(keys, axis=0)
        top = jnp.zeros((_TH, w), f32)
        for _ in range(_TOPK):
            mx = jnp.max(kk, axis=0)
            top = top + jax.lax.bitcast_convert_type(mx & jnp.int32(-64), f32)
            kk = jnp.where(kk == mx[None], jnp.int32(-1), kk)
        cs = cs + jnp.sum((tot - top) * colmask)

    irows = [xb[ch * _NV + _CTR, 3:12, :] for ch in range(3)]
    gx_abs = sum(
        jnp.abs(ir[0:8, 5:5 + w] - ir[0:8, 4:4 + w]) for ir in irows)
    wxv = jnp.exp(-50.0 * gx_abs)
    dgx = p9[0:8, 5:5 + w] - p9[0:8, 4:4 + w]
    mgx = ((ciota >= 8.0) & (ciota <= float(w - 10))).astype(f32)
    gxs = jnp.sum(wxv * jnp.abs(dgx) * mgx)
    gy_abs = sum(
        jnp.abs(ir[1:9, 4:4 + w] - ir[0:8, 4:4 + w]) for ir in irows)
    wyv = jnp.exp(-50.0 * gy_abs)
    dgy = p9[1:9, 4:4 + w] - p9[0:8, 4:4 + w]
    mgy = ((ciota >= 8.0) & (ciota < float(w - 8))
           & (yy <= float(h - 10))).astype(f32)
    gys = jnp.sum(wyv * jnp.abs(dgy) * mgy)

    ri = jax.lax.broadcasted_iota(jnp.int32, (8, 128), 0)
    ci = jax.lax.broadcasted_iota(jnp.int32, (8, 128), 1)
    contrib = (jnp.where((ri == 0) & (ci == 0), cs, 0.0)
               + jnp.where((ri == 0) & (ci == 1), gxs, 0.0)
               + jnp.where((ri == 0) & (ci == 2), gys, 0.0))

    @pl.when(first)
    def _init():
        out_ref[:] = jnp.zeros((8, 128), f32)

    out_ref[:] = out_ref[:] + contrib


def kernel(pred, x, epoch):
    b, u, v, h, w, c = x.shape
    xr = jnp.transpose(x.reshape(b, u * v, h, w, c), (0, 4, 1, 2, 3))
    xr = xr.reshape(b, c * u * v, h, w)
    xp = jnp.pad(xr, ((0, 0), (0, 0), (3, 5), (4, 4)))
    pp = jnp.pad(pred, ((0, 0), (3, 5), (4, 4)))

    k_t = jnp.where(epoch < 200, 0,
                    jnp.where(epoch < 2300, (epoch - 200) // 100 * 2, 44))
    m_t = (49 - k_t).astype(jnp.float32)

    ntiles = (h - 16) // _TH
    sums = pl.pallas_call(
        functools.partial(_body, h, w),
        grid=(b, ntiles),
        in_specs=[
            pl.BlockSpec((1, 3 * _NV, _TH, w + 8), lambda bi, t: (bi, 0, t + 1, 0)),
            pl.BlockSpec((1, 3 * _NV, _TH, w + 8), lambda bi, t: (bi, 0, t + 2, 0)),
            pl.BlockSpec((1, h + 8, w + 8), lambda bi, t: (bi, 0, 0)),
        ],
        out_specs=pl.BlockSpec((8, 128), lambda bi, t: (0, 0)),
        out_shape=jax.ShapeDtypeStruct((8, 128), jnp.float32),
    )(xp, xp, pp)

    color = sums[0, 0] * (49.0 / m_t) / (b * 3 * _NV * (h - 16) * (w - 16))
    grad = (sums[0, 1] / (b * (h - 16) * (w - 17))
            + sums[0, 2] / (b * (h - 17) * (w - 16))) * 0.5
    return color + 0.1 * grad
